# baseline (speedup 1.0000x reference)
import functools

import jax
import jax.numpy as jnp
from jax import lax
from jax.experimental import pallas as pl
from jax.experimental.pallas import tpu as pltpu
from jax.experimental.pallas import tpu_sc as plsc

_L, _B, _H, _N = 2, 4, 12, 577
_K = int(_N * 0.5)
_NP = _N - 1
_LANES = 16
_NCHUNK = _NP // _LANES
_WPAD = 592
_RPT = 3

_T0 = 18
_H0 = _T0 // 2
_NH = _H - _H0
_P = _B * _NH
_TCG = _B * _H0
_MCH = 64

_RH = 296
_RV = _N - _RH


def _tc_scores_body(rgb_lo_ref, rgb_hi_ref, tir_lo_ref, tir_hi_ref,
                    vrgb_ref, vtir_ref, s_ref):
    dn = (((1,), (0,)), ((), ()))

    def scores(v_ref, lo_ref, hi_ref):
        v = v_ref[0, 0, 0, 0:1, :]
        s = jax.lax.dot_general(v[:, :_RH], lo_ref[0, 0, 0], dn,
                                preferred_element_type=jnp.float32)
        s += jax.lax.dot_general(v[:, _RH:], hi_ref[0, 0, 0, :_RV, :], dn,
                                 preferred_element_type=jnp.float32)
        return s

    s_ref[0, 0, :] = scores(vrgb_ref, rgb_lo_ref, rgb_hi_ref)[0, 1:]
    s_ref[1, 0, :] = scores(vtir_ref, tir_lo_ref, tir_hi_ref)[0, 1:]


def _sc_scores_kernel(rgb_hbm, tir_hbm, out_hbm, vrow_v, mbuf_v, mtail_v,
                      acc_v, acc2_v):
    c = lax.axis_index("c")
    s = lax.axis_index("s")
    lane = lax.iota(jnp.int32, _LANES)
    fzeros = jnp.zeros((_LANES,), jnp.float32)

    def accum(splat, i, buf):
        for ch in range(_N // _LANES):
            sl = pl.ds(ch * _LANES, _LANES)
            acc_v[sl] = acc_v[sl] + splat * buf[i, sl]
        acc2_v[...] = acc2_v[...] + splat * buf[i, pl.ds(_N - _LANES, _LANES)]

    def do_items(attn_ref, mod):
        def rep_body(rep, _):
            p = s + 16 * rep

            @pl.when(p < _P)
            def _():
                b = p // _NH
                h = _H0 + p % _NH
                pltpu.sync_copy(attn_ref.at[1, b, h, pl.ds(0, 8)], vrow_v)
                for ch in range(_WPAD // _LANES):
                    acc_v[pl.ds(ch * _LANES, _LANES)] = fzeros
                acc2_v[...] = fzeros

                def chunk(c8, _):
                    pltpu.sync_copy(
                        attn_ref.at[0, b, h, pl.ds(c8 * _MCH, _MCH)], mbuf_v)

                    def row(i, _):
                        gi = c8 * _MCH + i
                        vchunk = vrow_v[0, pl.ds((gi >> 4) * _LANES, _LANES)]
                        li = jnp.full((_LANES,), 0, jnp.int32) + (gi & (_LANES - 1))
                        accum(vchunk.at[li].get(mode="promise_in_bounds"), i,
                              mbuf_v)
                        return 0

                    lax.fori_loop(0, _MCH, row, 0)
                    return 0

                lax.fori_loop(0, _N // _MCH, chunk, 0)
                pltpu.sync_copy(attn_ref.at[0, b, h, pl.ds(_N - 1, 1)],
                                mtail_v)
                vtail = vrow_v[0, pl.ds(_N - _LANES, _LANES)]
                accum(vtail.at[jnp.full((_LANES,), _LANES - 1, jnp.int32)]
                      .get(mode="promise_in_bounds"), 0, mtail_v)
                t36 = acc2_v[...].at[
                    jnp.full((_LANES,), _LANES - 1, jnp.int32)].get(
                        mode="promise_in_bounds")
                acc_v[pl.ds(_NP, _LANES)] = t36
                pltpu.sync_copy(acc_v, out_hbm.at[mod, p])

            return 0

        lax.fori_loop(0, (_P + 15) // 16, rep_body, 0)

    @pl.when(c == 0)
    def _():
        do_items(rgb_hbm, 0)

    @pl.when(c == 1)
    def _():
        do_items(tir_hbm, 1)


_sc_scores = functools.partial(
    pl.kernel,
    mesh=plsc.VectorSubcoreMesh(core_axis_name="c", subcore_axis_name="s"),
    out_type=jax.ShapeDtypeStruct((_L, _P, _WPAD), jnp.float32),
    scratch_types=[
        pltpu.VMEM((8, _N), jnp.float32),
        pltpu.VMEM((_MCH, _N), jnp.float32),
        pltpu.VMEM((1, _N), jnp.float32),
        pltpu.VMEM((_WPAD,), jnp.float32),
        pltpu.VMEM((_LANES,), jnp.float32),
    ],
)(_sc_scores_kernel)


def _sc_mask_kernel(tc_hbm, sc_hbm, out_hbm, floats_v, raw_v, mask_v, acc_v,
                    onehot_v, shared_rows):
    c = lax.axis_index("c")
    s = lax.axis_index("s")
    zeros = jnp.zeros((_LANES,), jnp.float32)
    ones = jnp.ones((_LANES,), jnp.float32)
    onei = jnp.full((_LANES,), 1, jnp.int32)
    zeroi = jnp.zeros((_LANES,), jnp.int32)
    lane = lax.iota(jnp.int32, _LANES)
    b_local = s // 8

    def topk_row_to_slot(slot):
        lo = jnp.zeros((_LANES,), jnp.float32)
        hi = jnp.full((_LANES,), 1024.0, jnp.float32)

        def bisect(_, carry):
            lo, hi = carry
            mid = 0.5 * (lo + hi)
            cnt = jnp.zeros((_LANES,), jnp.int32)
            for ch in range(_NCHUNK):
                m = floats_v[pl.ds(ch * _LANES, _LANES)] >= mid
                cnt = cnt + jnp.where(m, onei, zeroi)
            for sh in (8, 4, 2, 1):
                idx = (lane + sh) & (_LANES - 1)
                cnt = cnt + cnt.at[idx].get(mode="promise_in_bounds")
            ge = cnt >= _K
            return jnp.where(ge, mid, lo), jnp.where(ge, hi, mid)

        lo, hi = lax.fori_loop(0, 50, bisect, (lo, hi))
        for ch in range(_NCHUNK):
            sl = pl.ds(ch * _LANES, _LANES)
            m = floats_v[sl] >= lo
            mask_v[sl] = jnp.where(m, ones, zeros)
        pltpu.sync_copy(mask_v, shared_rows.at[slot])

    @pl.when((s % 8) < _T0 // _RPT)
    def _():
        for j in range(_RPT):
            r = _RPT * s + j
            t = r % 24
            gt = (2 * c + b_local) * _T0 + t
            pltpu.sync_copy(tc_hbm.at[gt, 0], floats_v)
            topk_row_to_slot(r)

    @pl.when((s % 8) >= _T0 // _RPT)
    def _():
        for j in range(_RPT):
            r = _RPT * s + j
            t = r % 24
            mod_t = (t - _T0) & 1
            hh = (t - _T0) >> 1
            q = (2 * c + b_local) * _NH + hh
            pltpu.sync_copy(sc_hbm.at[mod_t, q], raw_v)
            idxp1 = (lane + 1) & (_LANES - 1)
            for ch in range(_NCHUNK):
                a = raw_v[pl.ds(ch * _LANES, _LANES)]
                bnext = raw_v[pl.ds((ch + 1) * _LANES, _LANES)]
                rot = a.at[idxp1].get(mode="promise_in_bounds")
                b0 = bnext.at[zeroi].get(mode="promise_in_bounds")
                floats_v[pl.ds(ch * _LANES, _LANES)] = jnp.where(
                    lane == (_LANES - 1), b0, rot)
            topk_row_to_slot(r)

    plsc.subcore_barrier()

    @pl.when(s < 2)
    def _():
        for ch in range(_NCHUNK):
            acc_v[pl.ds(ch * _LANES, _LANES)] = zeros
        for r in range(_H * 2):
            pltpu.sync_copy(shared_rows.at[s * _H * 2 + r], mask_v)
            for ch in range(_NCHUNK):
                sl = pl.ds(ch * _LANES, _LANES)
                acc_v[sl] = acc_v[sl] + mask_v[sl]
        for ch in range(_NCHUNK):
            sl = pl.ds(ch * _LANES, _LANES)
            t = jnp.where(acc_v[sl] > 0.0, ones, zeros)
            onehot_v[0, sl] = ones - t
            onehot_v[1, sl] = t
        tail = pl.ds(_NCHUNK * _LANES, _LANES)
        onehot_v[0, tail] = ones
        onehot_v[1, tail] = zeros
        pltpu.sync_copy(onehot_v, out_hbm.at[2 * c + s])


_sc_mask = functools.partial(
    pl.kernel,
    mesh=plsc.VectorSubcoreMesh(core_axis_name="c", subcore_axis_name="s"),
    out_type=jax.ShapeDtypeStruct((_B, 2, _WPAD), jnp.float32),
    scratch_types=[
        pltpu.VMEM((_NP,), jnp.float32),
        pltpu.VMEM((_WPAD,), jnp.float32),
        pltpu.VMEM((_NP,), jnp.float32),
        pltpu.VMEM((_NP,), jnp.float32),
        pltpu.VMEM((2, _WPAD), jnp.float32),
        pltpu.VMEM_SHARED((48, _NP), jnp.float32),
    ],
)(_sc_mask_kernel)


def kernel(RGB_attn, TIR_attn):
    sc_scores = _sc_scores(RGB_attn, TIR_attn)

    tc_scores = pl.pallas_call(
        _tc_scores_body,
        grid=(_TCG,),
        in_specs=[
            pl.BlockSpec((1, 1, 1, _RH, _N), lambda i: (0, i // _H0, i % _H0, 0, 0)),
            pl.BlockSpec((1, 1, 1, _RH, _N), lambda i: (0, i // _H0, i % _H0, 1, 0)),
            pl.BlockSpec((1, 1, 1, _RH, _N), lambda i: (0, i // _H0, i % _H0, 0, 0)),
            pl.BlockSpec((1, 1, 1, _RH, _N), lambda i: (0, i // _H0, i % _H0, 1, 0)),
            pl.BlockSpec((1, 1, 1, 8, _N), lambda i: (1, i // _H0, i % _H0, 0, 0)),
            pl.BlockSpec((1, 1, 1, 8, _N), lambda i: (1, i // _H0, i % _H0, 0, 0)),
        ],
        out_specs=pl.BlockSpec((2, 1, _NP), lambda i: (i, 0, 0)),
        out_shape=jax.ShapeDtypeStruct((_B * _T0, 1, _NP), jnp.float32),
        compiler_params=pltpu.CompilerParams(
            dimension_semantics=("arbitrary",),
        ),
    )(RGB_attn, RGB_attn, TIR_attn, TIR_attn, RGB_attn, TIR_attn)

    out = _sc_mask(tc_scores, sc_scores)
    return jnp.transpose(out[:, :, :_N], (0, 2, 1))


# speedup vs baseline: 1.3892x; 1.3892x over previous
import functools

import jax
import jax.numpy as jnp
from jax import lax
from jax.experimental import pallas as pl
from jax.experimental.pallas import tpu as pltpu
from jax.experimental.pallas import tpu_sc as plsc

_L, _B, _H, _N = 2, 4, 12, 577
_G = _B * _H
_K = int(_N * 0.5)
_NP = _N - 1
_NROWS = 2 * _G
_LANES = 16
_NCHUNK = _NP // _LANES
_WPAD = 592
_RPT = 3

_RH = 296
_RV = _N - _RH


def _scores_body(rgb_lo_ref, rgb_hi_ref, tir_lo_ref, tir_hi_ref,
                 vrgb_ref, vtir_ref, s_ref):
    dn = (((1,), (0,)), ((), ()))

    def scores(v_ref, lo_ref, hi_ref):
        v = v_ref[0, 0, 0, 0:1, :]
        s = jax.lax.dot_general(v[:, :_RH], lo_ref[0, 0, 0], dn,
                                preferred_element_type=jnp.float32)
        s += jax.lax.dot_general(v[:, _RH:], hi_ref[0, 0, 0, :_RV, :], dn,
                                 preferred_element_type=jnp.float32)
        return s

    s_ref[0, 0, :] = jax.lax.bitcast_convert_type(
        scores(vrgb_ref, rgb_lo_ref, rgb_hi_ref)[0, 1:], jnp.int32)
    s_ref[1, 0, :] = jax.lax.bitcast_convert_type(
        scores(vtir_ref, tir_lo_ref, tir_hi_ref)[0, 1:], jnp.int32)


def _sc_mask_kernel(scores_hbm, out_hbm, row_v, ints_v, mask_v, acc_v,
                    onehot_v, shared_rows):
    c = lax.axis_index("c")
    s = lax.axis_index("s")
    zeros = jnp.zeros((_LANES,), jnp.float32)
    ones = jnp.ones((_LANES,), jnp.float32)

    g0 = (c * 16 + s) * _RPT

    onei = jnp.full((_LANES,), 1, jnp.int32)
    zeroi = jnp.zeros((_LANES,), jnp.int32)
    lane = lax.iota(jnp.int32, _LANES)

    for j in range(_RPT):
        g = g0 + j
        pltpu.sync_copy(scores_hbm.at[g, 0], ints_v)

        lo = jnp.zeros((_LANES,), jnp.int32)
        hi = jnp.full((_LANES,), 0x7F800000, jnp.int32)

        def bisect(_, carry):
            lo, hi = carry
            mid = lo + ((hi - lo) >> 1)
            cnt = jnp.zeros((_LANES,), jnp.int32)
            for ch in range(_NCHUNK):
                m = ints_v[pl.ds(ch * _LANES, _LANES)] >= mid
                cnt = cnt + jnp.where(m, onei, zeroi)
            for sh in (8, 4, 2, 1):
                idx = (lane + sh) & (_LANES - 1)
                cnt = cnt + cnt.at[idx].get(mode="promise_in_bounds")
            ge = cnt >= _K
            return jnp.where(ge, mid, lo), jnp.where(ge, hi, mid)

        lo, hi = lax.fori_loop(0, 31, bisect, (lo, hi))

        for ch in range(_NCHUNK):
            sl = pl.ds(ch * _LANES, _LANES)
            m = ints_v[sl] >= lo
            mask_v[sl] = jnp.where(m, ones, zeros)
        pltpu.sync_copy(mask_v, shared_rows.at[s * _RPT + j])

    plsc.subcore_barrier()

    @pl.when(s < 2)
    def _():
        for ch in range(_NCHUNK):
            acc_v[pl.ds(ch * _LANES, _LANES)] = zeros
        for r in range(_H * 2):
            pltpu.sync_copy(shared_rows.at[s * _H * 2 + r], row_v)
            for ch in range(_NCHUNK):
                sl = pl.ds(ch * _LANES, _LANES)
                acc_v[sl] = acc_v[sl] + row_v[sl]
        for ch in range(_NCHUNK):
            sl = pl.ds(ch * _LANES, _LANES)
            t = jnp.where(acc_v[sl] > 0.0, ones, zeros)
            onehot_v[0, sl] = ones - t
            onehot_v[1, sl] = t
        tail = pl.ds(_NCHUNK * _LANES, _LANES)
        onehot_v[0, tail] = ones
        onehot_v[1, tail] = zeros
        pltpu.sync_copy(onehot_v, out_hbm.at[2 * c + s])


_sc_mask = functools.partial(
    pl.kernel,
    mesh=plsc.VectorSubcoreMesh(core_axis_name="c", subcore_axis_name="s"),
    out_type=jax.ShapeDtypeStruct((_B, 2, _WPAD), jnp.float32),
    scratch_types=[
        pltpu.VMEM((_NP,), jnp.float32),
        pltpu.VMEM((_NP,), jnp.int32),
        pltpu.VMEM((_NP,), jnp.float32),
        pltpu.VMEM((_NP,), jnp.float32),
        pltpu.VMEM((2, _WPAD), jnp.float32),
        pltpu.VMEM_SHARED((_G, _NP), jnp.float32),
    ],
)(_sc_mask_kernel)


def kernel(RGB_attn, TIR_attn):
    scores = pl.pallas_call(
        _scores_body,
        grid=(_G,),
        in_specs=[
            pl.BlockSpec((1, 1, 1, _RH, _N), lambda i: (0, i // _H, i % _H, 0, 0)),
            pl.BlockSpec((1, 1, 1, _RH, _N), lambda i: (0, i // _H, i % _H, 1, 0)),
            pl.BlockSpec((1, 1, 1, _RH, _N), lambda i: (0, i // _H, i % _H, 0, 0)),
            pl.BlockSpec((1, 1, 1, _RH, _N), lambda i: (0, i // _H, i % _H, 1, 0)),
            pl.BlockSpec((1, 1, 1, 8, _N), lambda i: (1, i // _H, i % _H, 0, 0)),
            pl.BlockSpec((1, 1, 1, 8, _N), lambda i: (1, i // _H, i % _H, 0, 0)),
        ],
        out_specs=pl.BlockSpec((2, 1, _NP), lambda i: (i, 0, 0)),
        out_shape=jax.ShapeDtypeStruct((_NROWS, 1, _NP), jnp.int32),
        compiler_params=pltpu.CompilerParams(
            dimension_semantics=("arbitrary",),
        ),
    )(RGB_attn, RGB_attn, TIR_attn, TIR_attn, RGB_attn, TIR_attn)

    out = _sc_mask(scores)
    return jnp.transpose(out[:, :, :_N], (0, 2, 1))


# speedup vs baseline: 1.4102x; 1.0151x over previous
import functools

import jax
import jax.numpy as jnp
from jax import lax
from jax.experimental import pallas as pl
from jax.experimental.pallas import tpu as pltpu
from jax.experimental.pallas import tpu_sc as plsc

_L, _B, _H, _N = 2, 4, 12, 577
_G = _B * _H
_K = int(_N * 0.5)
_NP = _N - 1
_NROWS = 2 * _G
_LANES = 16
_NCHUNK = _NP // _LANES
_WPAD = 592
_RPT = 3

_RH = 296
_RV = _N - _RH


def _scores_body(rgb_lo_ref, rgb_hi_ref, tir_lo_ref, tir_hi_ref,
                 vrgb_ref, vtir_ref, s_ref):
    dn = (((1,), (0,)), ((), ()))

    def scores(v_ref, lo_ref, hi_ref):
        v = v_ref[0, 0, 0, 0:1, :]
        s = jax.lax.dot_general(v[:, :_RH], lo_ref[0, 0, 0], dn,
                                preferred_element_type=jnp.float32)
        s += jax.lax.dot_general(v[:, _RH:], hi_ref[0, 0, 0, :_RV, :], dn,
                                 preferred_element_type=jnp.float32)
        return s

    s_ref[0, 0, :] = jax.lax.bitcast_convert_type(
        scores(vrgb_ref, rgb_lo_ref, rgb_hi_ref)[0, 1:], jnp.int32)
    s_ref[1, 0, :] = jax.lax.bitcast_convert_type(
        scores(vtir_ref, tir_lo_ref, tir_hi_ref)[0, 1:], jnp.int32)


def _sc_mask_kernel(scores_hbm, out_hbm, ints3_v, mask3_v, rows24_v, acc_v,
                    onehot_v, shared_rows):
    c = lax.axis_index("c")
    s = lax.axis_index("s")
    zeros = jnp.zeros((_LANES,), jnp.float32)
    ones = jnp.ones((_LANES,), jnp.float32)

    g0 = (c * 16 + s) * _RPT

    onei = jnp.full((_LANES,), 1, jnp.int32)
    zeroi = jnp.zeros((_LANES,), jnp.int32)
    lane = lax.iota(jnp.int32, _LANES)

    pltpu.sync_copy(scores_hbm.at[pl.ds(g0, _RPT)], ints3_v)
    for j in range(_RPT):
        lo = jnp.zeros((_LANES,), jnp.int32)
        hi = jnp.full((_LANES,), 0x7F800000, jnp.int32)

        def bisect(_, carry):
            lo, hi = carry
            mid = lo + ((hi - lo) >> 1)
            cnt = jnp.zeros((_LANES,), jnp.int32)
            for ch in range(_NCHUNK):
                m = ints3_v[j, 0, pl.ds(ch * _LANES, _LANES)] >= mid
                cnt = cnt + jnp.where(m, onei, zeroi)
            for sh in (8, 4, 2, 1):
                idx = (lane + sh) & (_LANES - 1)
                cnt = cnt + cnt.at[idx].get(mode="promise_in_bounds")
            ge = cnt >= _K
            return jnp.where(ge, mid, lo), jnp.where(ge, hi, mid)

        lo, hi = lax.fori_loop(0, 31, bisect, (lo, hi))

        for ch in range(_NCHUNK):
            sl = pl.ds(ch * _LANES, _LANES)
            m = ints3_v[j, 0, sl] >= lo
            mask3_v[j, 0, sl] = jnp.where(m, ones, zeros)
    pltpu.sync_copy(mask3_v, shared_rows.at[pl.ds(s * _RPT, _RPT)])

    plsc.subcore_barrier()

    @pl.when(s < 2)
    def _():
        pltpu.sync_copy(shared_rows.at[pl.ds(s * _H * 2, _H * 2)], rows24_v)
        for ch in range(_NCHUNK):
            sl = pl.ds(ch * _LANES, _LANES)
            acc = rows24_v[0, 0, sl]
            for r in range(1, _H * 2):
                acc = acc + rows24_v[r, 0, sl]
            t = jnp.where(acc > 0.0, ones, zeros)
            onehot_v[0, sl] = ones - t
            onehot_v[1, sl] = t
        tail = pl.ds(_NCHUNK * _LANES, _LANES)
        onehot_v[0, tail] = ones
        onehot_v[1, tail] = zeros
        pltpu.sync_copy(onehot_v, out_hbm.at[2 * c + s])


_sc_mask = functools.partial(
    pl.kernel,
    mesh=plsc.VectorSubcoreMesh(core_axis_name="c", subcore_axis_name="s"),
    out_type=jax.ShapeDtypeStruct((_B, 2, _WPAD), jnp.float32),
    scratch_types=[
        pltpu.VMEM((_RPT, 1, _NP), jnp.int32),
        pltpu.VMEM((_RPT, 1, _NP), jnp.float32),
        pltpu.VMEM((_H * 2, 1, _NP), jnp.float32),
        pltpu.VMEM((_NP,), jnp.float32),
        pltpu.VMEM((2, _WPAD), jnp.float32),
        pltpu.VMEM_SHARED((_G, 1, _NP), jnp.float32),
    ],
)(_sc_mask_kernel)


def kernel(RGB_attn, TIR_attn):
    scores = pl.pallas_call(
        _scores_body,
        grid=(_G,),
        in_specs=[
            pl.BlockSpec((1, 1, 1, _RH, _N), lambda i: (0, i // _H, i % _H, 0, 0)),
            pl.BlockSpec((1, 1, 1, _RH, _N), lambda i: (0, i // _H, i % _H, 1, 0)),
            pl.BlockSpec((1, 1, 1, _RH, _N), lambda i: (0, i // _H, i % _H, 0, 0)),
            pl.BlockSpec((1, 1, 1, _RH, _N), lambda i: (0, i // _H, i % _H, 1, 0)),
            pl.BlockSpec((1, 1, 1, 8, _N), lambda i: (1, i // _H, i % _H, 0, 0)),
            pl.BlockSpec((1, 1, 1, 8, _N), lambda i: (1, i // _H, i % _H, 0, 0)),
        ],
        out_specs=pl.BlockSpec((2, 1, _NP), lambda i: (i, 0, 0)),
        out_shape=jax.ShapeDtypeStruct((_NROWS, 1, _NP), jnp.int32),
        compiler_params=pltpu.CompilerParams(
            dimension_semantics=("arbitrary",),
        ),
    )(RGB_attn, RGB_attn, TIR_attn, TIR_attn, RGB_attn, TIR_attn)

    out = _sc_mask(scores)
    return jnp.transpose(out[:, :, :_N], (0, 2, 1))
